# baseline (speedup 1.0000x reference)
import functools

import jax
import jax.numpy as jnp
from jax import lax
from jax.experimental import pallas as pl
from jax.experimental.pallas import tpu as pltpu
from jax.experimental.pallas import tpu_sc as plsc

B = 4096
D = 16


def _build_gather():
    info = plsc.get_sparse_core_info()
    nc, ns = info.num_cores, info.num_subcores
    nw = nc * ns
    bpw = B // nw
    mesh = plsc.VectorSubcoreMesh(core_axis_name="c", subcore_axis_name="s")

    @functools.partial(
        pl.kernel,
        mesh=mesh,
        out_type=[
            jax.ShapeDtypeStruct((B, D), jnp.float32),
            jax.ShapeDtypeStruct((B, D), jnp.float32),
        ],
        scratch_types=[
            pltpu.VMEM((bpw,), jnp.int32),
            pltpu.VMEM((bpw, D), jnp.float32),
            pltpu.VMEM((bpw,), jnp.int32),
            pltpu.VMEM((bpw, D), jnp.float32),
            pltpu.SemaphoreType.DMA,
            pltpu.SemaphoreType.DMA,
        ],
        compiler_params=pltpu.CompilerParams(use_tc_tiling_on_sc=False),
    )
    def gather2(tgt_hbm, ctx_hbm, table_hbm, t_out, c_out,
                tidx_v, trows_v, cidx_v, crows_v, sem_t, sem_c):
        wid = lax.axis_index("s") * nc + lax.axis_index("c")
        base = wid * bpw
        pltpu.sync_copy(tgt_hbm.at[pl.ds(base, bpw)], tidx_v)
        pltpu.sync_copy(ctx_hbm.at[pl.ds(base, bpw)], cidx_v)
        cp_t = pltpu.async_copy(table_hbm.at[tidx_v], trows_v, sem_t)
        cp_c = pltpu.async_copy(table_hbm.at[cidx_v], crows_v, sem_c)
        cp_t.wait()
        cp_c.wait()
        pltpu.sync_copy(trows_v, t_out.at[pl.ds(base, bpw)])
        pltpu.sync_copy(crows_v, c_out.at[pl.ds(base, bpw)])

    return gather2


_gather2 = _build_gather()

_BM = 512


def _mm_body(t_ref, c_ref, o_ref):
    o_ref[...] = lax.dot_general(
        t_ref[...], c_ref[...],
        dimension_numbers=(((1,), (1,)), ((), ())),
        preferred_element_type=jnp.float32,
    )


_matmul = pl.pallas_call(
    _mm_body,
    grid=(B // _BM,),
    in_specs=[
        pl.BlockSpec((_BM, D), lambda i: (i, 0)),
        pl.BlockSpec((B, D), lambda i: (0, 0)),
    ],
    out_specs=pl.BlockSpec((_BM, B), lambda i: (i, 0)),
    out_shape=jax.ShapeDtypeStruct((B, B), jnp.float32),
)


def kernel(target, context, table):
    target = target.astype(jnp.int32)
    context = context.astype(jnp.int32)
    t_emb, c_emb = _gather2(target, context, table)
    return _matmul(t_emb, c_emb)
